# Initial kernel scaffold; baseline (speedup 1.0000x reference)
import jax
import jax.numpy as jnp
from jax.experimental import pallas as pl


def kernel(xyz, params):
    raise NotImplementedError("write your pallas kernel here")


# speedup vs baseline: 3.5858x; 3.5858x over previous
import functools

import jax
import jax.numpy as jnp
from jax.experimental import pallas as pl


def _fps_kernel(xyzt_ref, out_ref, *, npoint):
    pts = xyzt_ref[0]
    N = pts.shape[1]
    x = pts[0:1, :]
    y = pts[1:2, :]
    z = pts[2:3, :]
    iota = jax.lax.broadcasted_iota(jnp.int32, (1, N), 1)
    iota_p = jax.lax.broadcasted_iota(jnp.int32, (1, npoint), 1)

    def body(i, carry):
        dist, far, out_vec = carry
        out_vec = jnp.where(iota_p == i, far, out_vec)
        oh = (iota == far).astype(jnp.float32)
        cx = jnp.sum(x * oh)
        cy = jnp.sum(y * oh)
        cz = jnp.sum(z * oh)
        d = (x - cx) ** 2 + (y - cy) ** 2 + (z - cz) ** 2
        dist = jnp.minimum(dist, d)
        mx = jnp.max(dist)
        far = jnp.min(jnp.where(dist == mx, iota, N))
        return dist, far, out_vec

    dist0 = jnp.full((1, N), 1e10, jnp.float32)
    out0 = jnp.zeros((1, npoint), jnp.int32)
    _, _, out_vec = jax.lax.fori_loop(
        0, npoint, body, (dist0, jnp.int32(0), out0))
    out_ref[0] = out_vec


def _ball_kernel(nxyz_ref, xyzt_ref, out_ref, *, radius2, nsample):
    s = nxyz_ref[0]
    xt = xyzt_ref[0]
    N = xt.shape[1]
    sn = jnp.sum(s * s, axis=1, keepdims=True)
    xn = jnp.sum(xt * xt, axis=0, keepdims=True)
    d = -2.0 * jnp.dot(s, xt, preferred_element_type=jnp.float32)
    d = d + sn
    d = d + xn
    keep = jnp.logical_not(d > radius2)
    iota = jax.lax.broadcasted_iota(jnp.int32, (1, N), 1)
    prev = jnp.full((s.shape[0], 1), -1, jnp.int32)
    cols = []
    for _ in range(nsample):
        cand = jnp.where(keep & (iota > prev), iota, N)
        nk = jnp.min(cand, axis=1, keepdims=True)
        cols.append(nk)
        prev = nk
    idx = jnp.concatenate(cols, axis=1)
    first = idx[:, :1]
    out_ref[0] = jnp.where(idx == N, first, idx)


def _knn3_kernel(d_ref, p2_ref, out_ref):
    d = d_ref[0]
    p2 = p2_ref[0]
    S2 = d.shape[1]
    iota = jax.lax.broadcasted_iota(jnp.int32, (1, S2), 1)
    dcur = d
    vals = []
    idxs = []
    for _ in range(3):
        mv = jnp.min(dcur, axis=1, keepdims=True)
        aj = jnp.min(jnp.where(dcur == mv, iota, S2), axis=1, keepdims=True)
        vals.append(mv)
        idxs.append(aj)
        dcur = jnp.where(iota == aj, 1e30, dcur)
    d3 = jnp.concatenate(vals, axis=1)
    recip = 1.0 / (d3 + 1e-8)
    norm = jnp.sum(recip, axis=1, keepdims=True)
    w = recip / norm
    m = jnp.zeros(d.shape, jnp.float32)
    for j in range(3):
        m = m + jnp.where(iota == idxs[j], w[:, j:j + 1], 0.0)
    out_ref[0] = jnp.dot(m, p2, precision=jax.lax.Precision.HIGHEST,
                         preferred_element_type=jnp.float32)


def _fps(xyz, npoint):
    B, N, _ = xyz.shape
    xt = jnp.transpose(xyz, (0, 2, 1))
    out = pl.pallas_call(
        functools.partial(_fps_kernel, npoint=npoint),
        grid=(B,),
        in_specs=[pl.BlockSpec((1, 3, N), lambda b: (b, 0, 0))],
        out_specs=pl.BlockSpec((1, 1, npoint), lambda b: (b, 0, 0)),
        out_shape=jax.ShapeDtypeStruct((B, 1, npoint), jnp.int32),
    )(xt)
    return out[:, 0, :]


def _ball(radius, nsample, xyz, new_xyz):
    B, N, _ = xyz.shape
    S = new_xyz.shape[1]
    sb = min(S, 128)
    xt = jnp.transpose(xyz, (0, 2, 1))
    out = pl.pallas_call(
        functools.partial(_ball_kernel, radius2=radius * radius,
                          nsample=nsample),
        grid=(B, S // sb),
        in_specs=[
            pl.BlockSpec((1, sb, 3), lambda b, s: (b, s, 0)),
            pl.BlockSpec((1, 3, N), lambda b, s: (b, 0, 0)),
        ],
        out_specs=pl.BlockSpec((1, sb, nsample), lambda b, s: (b, s, 0)),
        out_shape=jax.ShapeDtypeStruct((B, S, nsample), jnp.int32),
    )(new_xyz, xt)
    return out


def _sqdist(src, dst):
    d = -2.0 * jnp.einsum('bnc,bmc->bnm', src, dst)
    d = d + jnp.sum(src ** 2, axis=-1)[:, :, None]
    d = d + jnp.sum(dst ** 2, axis=-1)[:, None, :]
    return d


def _knn3_interp(xyz1, xyz2, points2):
    B, N1, _ = xyz1.shape
    S2 = xyz2.shape[1]
    C2 = points2.shape[2]
    nb = min(N1, 512)
    x1b, x2b = jax.lax.optimization_barrier((xyz1, xyz2))
    dists = jax.lax.optimization_barrier(_sqdist(x1b, x2b))
    out = pl.pallas_call(
        _knn3_kernel,
        grid=(B, N1 // nb),
        in_specs=[
            pl.BlockSpec((1, nb, S2), lambda b, n: (b, n, 0)),
            pl.BlockSpec((1, S2, C2), lambda b, n: (b, 0, 0)),
        ],
        out_specs=pl.BlockSpec((1, nb, C2), lambda b, n: (b, n, 0)),
        out_shape=jax.ShapeDtypeStruct((B, N1, C2), jnp.float32),
    )(dists, points2)
    return out


def _gather(points, idx):
    return jax.vmap(lambda p, i: p[i])(points, idx)


def _mlp(x, layers, axes):
    for (W, b, g, bt) in layers:
        x = x @ W + b
        m = jnp.mean(x, axis=axes, keepdims=True)
        v = jnp.var(x, axis=axes, keepdims=True)
        x = (x - m) / jnp.sqrt(v + 1e-5) * g + bt
        x = jax.nn.relu(x)
    return x


def _set_abstraction(xyz, points, npoint, radius, nsample, layers):
    fps_idx = _fps(xyz, npoint)
    new_xyz = _gather(xyz, fps_idx)
    idx = _ball(radius, nsample, xyz, new_xyz)
    grouped_xyz = _gather(xyz, idx)
    grouped_norm = grouped_xyz - new_xyz[:, :, None, :]
    grouped_points = _gather(points, idx)
    new_points = jnp.concatenate([grouped_norm, grouped_points], axis=-1)
    new_points = _mlp(new_points, layers, (0, 1, 2))
    return new_xyz, jnp.max(new_points, axis=2)


def _feature_propagation(xyz1, xyz2, points1, points2, layers):
    interpolated = _knn3_interp(xyz1, xyz2, points2)
    if points1 is not None:
        new_points = jnp.concatenate([points1, interpolated], axis=-1)
    else:
        new_points = interpolated
    return _mlp(new_points, layers, (0, 1))


def kernel(xyz, params):
    bar = jax.lax.optimization_barrier
    l0_points = jnp.transpose(xyz, (0, 2, 1))
    l0_xyz = l0_points[:, :, :3]
    l0_xyz, l0_points = bar((l0_xyz, l0_points))
    l1_xyz, l1_points = bar(_set_abstraction(l0_xyz, l0_points, 1024, 0.1,
                                             32, params['sa1']))
    l2_xyz, l2_points = bar(_set_abstraction(l1_xyz, l1_points, 256, 0.2,
                                             32, params['sa2']))
    l3_xyz, l3_points = bar(_set_abstraction(l2_xyz, l2_points, 64, 0.4,
                                             32, params['sa3']))
    l4_xyz, l4_points = bar(_set_abstraction(l3_xyz, l3_points, 16, 0.8,
                                             32, params['sa4']))
    l3_points = bar(_feature_propagation(l3_xyz, l4_xyz, l3_points,
                                         l4_points, params['fp4']))
    l2_points = bar(_feature_propagation(l2_xyz, l3_xyz, l2_points,
                                         l3_points, params['fp3']))
    l1_points = bar(_feature_propagation(l1_xyz, l2_xyz, l1_points,
                                         l2_points, params['fp2']))
    l0_feats = bar(_feature_propagation(l0_xyz, l1_xyz, None, l1_points,
                                        params['fp1']))
    h = _mlp(l0_feats, params['conv1'], (0, 1))
    W2, b2 = params['conv2']
    x = h @ W2 + b2
    Wm, bm = params['head_mlp']
    f = jax.nn.leaky_relu(x @ Wm + bm, 0.1)
    Wc, bc = params['head_center']
    c = jax.nn.sigmoid(f @ Wc + bc)
    Wv, bv = params['head_var']
    v = jax.nn.relu(jax.nn.leaky_relu(f @ Wv + bv, 0.1))
    Ws, bs = params['head_softmax']
    xo = jax.nn.leaky_relu(f @ Ws + bs, 0.1)
    return (xo, c, v, f)
